# Initial kernel scaffold; baseline (speedup 1.0000x reference)
import jax
import jax.numpy as jnp
from jax.experimental import pallas as pl


def kernel(pred, target):
    raise NotImplementedError("write your pallas kernel here")


# speedup vs baseline: 1.3407x; 1.3407x over previous
import jax
import jax.numpy as jnp
from jax.experimental import pallas as pl
from jax.experimental.pallas import tpu as pltpu

_BINS = 30
_ALPHA = 0.5
_ROWS = 512


def _ghm_kernel(pred_ref, tgt_ref, out_ref, cnt_ref, sum_ref):
    i = pl.program_id(0)
    ni = pl.num_programs(0)

    @pl.when(i == 0)
    def _init():
        cnt_ref[...] = jnp.zeros_like(cnt_ref)
        sum_ref[...] = jnp.zeros_like(sum_ref)

    x = pred_ref[...]
    t = tgt_ref[...]
    R, C = x.shape

    col = jax.lax.broadcasted_iota(jnp.int32, (R, C), 1)
    m = jnp.max(x, axis=1, keepdims=True)
    e = jnp.exp(x - m)
    s = jnp.sum(e, axis=1, keepdims=True)
    xt = jnp.sum(jnp.where(col == t, x, 0.0), axis=1, keepdims=True)
    logz = m + jnp.log(s)
    bl = logz - xt
    p = jnp.exp(xt - m) / s
    g = 1.0 - p
    b = jnp.clip(jnp.floor(g * _BINS).astype(jnp.int32), 0, _BINS - 1)

    lane = jax.lax.broadcasted_iota(jnp.int32, (R, 128), 1)
    onehot = (lane == b).astype(jnp.float32)
    cnt_ref[...] += jnp.sum(onehot, axis=0, keepdims=True)
    sum_ref[...] += jnp.sum(onehot * bl, axis=0, keepdims=True)

    @pl.when(i == ni - 1)
    def _fin():
        c = cnt_ref[...]
        S = sum_ref[...]
        out_ref[...] = _ALPHA * jnp.sum(S / (c + 1e-6), axis=1, keepdims=True)


def kernel(pred, target):
    n, c = pred.shape
    grid = n // _ROWS
    t2 = target.reshape(n, 1)
    out = pl.pallas_call(
        _ghm_kernel,
        grid=(grid,),
        in_specs=[
            pl.BlockSpec((_ROWS, c), lambda i: (i, 0)),
            pl.BlockSpec((_ROWS, 1), lambda i: (i, 0)),
        ],
        out_specs=pl.BlockSpec((1, 1), lambda i: (0, 0)),
        out_shape=jax.ShapeDtypeStruct((1, 1), jnp.float32),
        scratch_shapes=[
            pltpu.VMEM((1, 128), jnp.float32),
            pltpu.VMEM((1, 128), jnp.float32),
        ],
        compiler_params=pltpu.CompilerParams(
            dimension_semantics=("arbitrary",),
        ),
    )(pred, t2)
    return out[0, 0]
